# baseline (speedup 1.0000x reference)
import functools

import jax
import jax.numpy as jnp
from jax import lax
from jax.experimental import pallas as pl
from jax.experimental.pallas import tpu as pltpu
from jax.experimental.pallas import tpu_sc as plsc

B, L, D = 4096, 200, 128
LANES = 16
PE_ROWS = 216
SLOT_ROWS = 200
NSLOT = 3
CROWS = 8


def kernel(inputs, pos_embedding):
    info = plsc.get_sparse_core_info()
    nc, ns = info.num_cores, info.num_subcores
    nw = nc * ns
    rows = B // nw

    mesh = plsc.VectorSubcoreMesh(core_axis_name="c", subcore_axis_name="s")

    @functools.partial(
        pl.kernel,
        mesh=mesh,
        out_type=jax.ShapeDtypeStruct((B, L, D), jnp.float32),
        scratch_types=[
            pltpu.VMEM((PE_ROWS, D), jnp.float32),
            pltpu.VMEM((128,), jnp.int32),
            pltpu.VMEM((80,), jnp.int32),
            pltpu.VMEM((L, D), jnp.float32),
            pltpu.VMEM_SHARED((ns, NSLOT, SLOT_ROWS, D), jnp.float32),
            pltpu.SemaphoreType.DMA,
            pltpu.SemaphoreType.DMA,
            pltpu.SemaphoreType.DMA,
            pltpu.SemaphoreType.DMA,
            pltpu.SemaphoreType.DMA,
            pltpu.SemaphoreType.DMA,
        ],
    )
    def sc_add(in_hbm, pe_hbm, out_hbm, pe_v, idx_a, idx_b, cbuf, sp,
               la0, la1, la2, st0, st1, st2):
        c = lax.axis_index("c")
        s = lax.axis_index("s")
        wid = s * nc + c
        base = wid * rows
        lsems = (la0, la1, la2)
        ssems = (st0, st1, st2)

        pltpu.sync_copy(pe_hbm.at[pl.ds(0, 208)], pe_v.at[pl.ds(0, 208)])
        zero = jnp.zeros((LANES,), jnp.float32)
        for rr in range(201, PE_ROWS):
            for j in range(D // LANES):
                pe_v[rr, pl.ds(j * LANES, LANES)] = zero

        for k in range(8):
            idx_a[pl.ds(k * LANES, LANES)] = (
                lax.iota(jnp.int32, LANES) + k * LANES)
        for k in range(4):
            idx_b[pl.ds(k * LANES, LANES)] = (
                lax.iota(jnp.int32, LANES) + 128 + k * LANES)
        idx_b[pl.ds(4 * LANES, LANES)] = jnp.minimum(
            lax.iota(jnp.int32, LANES) + 192, 199)

        def data(k):
            return sp.at[s, k]

        def load(k, u):
            pltpu.async_copy(in_hbm.at[base + u], data(k), lsems[k])

        def wait_load(k, u):
            pltpu.make_async_copy(in_hbm.at[base + u], data(k),
                                  lsems[k]).wait()

        def store(k, u):
            pltpu.async_copy(data(k), out_hbm.at[base + u], ssems[k])

        def wait_store(k, u):
            pltpu.make_async_copy(data(k), out_hbm.at[base + u],
                                  ssems[k]).wait()

        def scatter_add(k):
            dst = sp.at[s, k]
            cp_a = pltpu.async_copy(
                pe_v.at[pl.ds(1, 128)], dst.at[idx_a], lsems[k], add=True)
            cp_b = pltpu.async_copy(
                pe_v.at[pl.ds(129, 80)], dst.at[idx_b], lsems[k], add=True)
            cp_a.wait()
            cp_b.wait()

        def process(k, u):
            wait_load(k, u)
            scatter_add(k)
            store(k, u)

        srows = rows - CROWS

        for k in range(NSLOT):
            load(k, k)

        def round3(t, carry):
            for k in range(NSLOT):
                process(k, 3 * t + k)

            @pl.when(jnp.logical_and(lax.rem(t, 5) == 0, t < 5 * CROWS))
            def _compute_row():
                crow = base + srows + t // 5
                pltpu.sync_copy(in_hbm.at[crow], cbuf)

                def cbody(l, carry2):
                    for j in range(D // LANES):
                        sl = pl.ds(j * LANES, LANES)
                        cbuf[l, sl] = cbuf[l, sl] + pe_v[l + 1, sl]
                    return carry2

                lax.fori_loop(0, L, cbody, 0)
                pltpu.sync_copy(cbuf, out_hbm.at[crow])

            for k in range(NSLOT):
                wait_store(k, 3 * t + k)
                load(k, 3 * t + 3 + k)
            return carry

        lax.fori_loop(0, srows // NSLOT - 1, round3, 0)

        for k in range(NSLOT):
            process(k, srows - NSLOT + k)
        for k in range(NSLOT):
            wait_store(k, srows - NSLOT + k)

    return sc_add(inputs, pos_embedding)


# speedup vs baseline: 1.0487x; 1.0487x over previous
import functools

import jax
import jax.numpy as jnp
from jax import lax
from jax.experimental import pallas as pl
from jax.experimental.pallas import tpu as pltpu
from jax.experimental.pallas import tpu_sc as plsc

B, L, D = 4096, 200, 128
LANES = 16
PE_ROWS = 216
SLOT_ROWS = 200
NSLOT = 4


def kernel(inputs, pos_embedding):
    info = plsc.get_sparse_core_info()
    nc, ns = info.num_cores, info.num_subcores
    nw = nc * ns
    rows = B // nw

    mesh = plsc.VectorSubcoreMesh(core_axis_name="c", subcore_axis_name="s")

    @functools.partial(
        pl.kernel,
        mesh=mesh,
        out_type=jax.ShapeDtypeStruct((B, L, D), jnp.float32),
        scratch_types=[
            pltpu.VMEM((PE_ROWS, D), jnp.float32),
            pltpu.VMEM((128,), jnp.int32),
            pltpu.VMEM((80,), jnp.int32),
            pltpu.VMEM_SHARED((ns, NSLOT, SLOT_ROWS, D), jnp.float32),
            pltpu.SemaphoreType.DMA,
            pltpu.SemaphoreType.DMA,
            pltpu.SemaphoreType.DMA,
            pltpu.SemaphoreType.DMA,
            pltpu.SemaphoreType.DMA,
            pltpu.SemaphoreType.DMA,
            pltpu.SemaphoreType.DMA,
            pltpu.SemaphoreType.DMA,
        ],
    )
    def sc_add(in_hbm, pe_hbm, out_hbm, pe_v, idx_a, idx_b, sp,
               la0, la1, la2, la3, st0, st1, st2, st3):
        c = lax.axis_index("c")
        s = lax.axis_index("s")
        wid = s * nc + c
        base = wid * rows
        lsems = (la0, la1, la2, la3)
        ssems = (st0, st1, st2, st3)

        pltpu.sync_copy(pe_hbm.at[pl.ds(0, 208)], pe_v.at[pl.ds(0, 208)])
        zero = jnp.zeros((LANES,), jnp.float32)
        for rr in range(201, PE_ROWS):
            for j in range(D // LANES):
                pe_v[rr, pl.ds(j * LANES, LANES)] = zero

        for k in range(8):
            idx_a[pl.ds(k * LANES, LANES)] = (
                lax.iota(jnp.int32, LANES) + k * LANES)
        for k in range(4):
            idx_b[pl.ds(k * LANES, LANES)] = (
                lax.iota(jnp.int32, LANES) + 128 + k * LANES)
        idx_b[pl.ds(4 * LANES, LANES)] = jnp.minimum(
            lax.iota(jnp.int32, LANES) + 192, 199)

        def data(k):
            return sp.at[s, k]

        def load(k, u):
            pltpu.async_copy(in_hbm.at[base + u], data(k), lsems[k])

        def wait_load(k, u):
            pltpu.make_async_copy(in_hbm.at[base + u], data(k),
                                  lsems[k]).wait()

        def store(k, u):
            pltpu.async_copy(data(k), out_hbm.at[base + u], ssems[k])

        def wait_store(k, u):
            pltpu.make_async_copy(data(k), out_hbm.at[base + u],
                                  ssems[k]).wait()

        def start_scatter(k):
            dst = sp.at[s, k]
            cp_a = pltpu.async_copy(
                pe_v.at[pl.ds(1, 128)], dst.at[idx_a], lsems[k], add=True)
            cp_b = pltpu.async_copy(
                pe_v.at[pl.ds(129, 80)], dst.at[idx_b], lsems[k], add=True)
            return cp_a, cp_b

        def process_pair(k0, u0, u1):
            wait_load(k0, u0)
            cps0 = start_scatter(k0)
            wait_load(k0 + 1, u1)
            cps1 = start_scatter(k0 + 1)
            for cp in cps0:
                cp.wait()
            store(k0, u0)
            for cp in cps1:
                cp.wait()
            store(k0 + 1, u1)

        for k in range(NSLOT):
            load(k, k)
        process_pair(0, 0, 1)

        def round2(t2, carry):
            r = 4 * t2
            for k in range(2):
                wait_store(k, r + k)
                load(k, r + 4 + k)
            process_pair(2, r + 2, r + 3)
            process_pair(0, r + 4, r + 5)
            for k in range(2):
                wait_store(2 + k, r + 2 + k)
                load(2 + k, r + 6 + k)
            return carry

        lax.fori_loop(0, (rows - 4) // 4, round2, 0)

        process_pair(2, rows - 2, rows - 1)
        for k in range(2):
            wait_store(k, rows - 4 + k)
        for k in range(2):
            wait_store(2 + k, rows - 2 + k)

    return sc_add(inputs, pos_embedding)
